# baseline (speedup 1.0000x reference)
import functools

import jax
import jax.numpy as jnp
from jax import lax
from jax.experimental import pallas as pl
from jax.experimental.pallas import tpu as pltpu
from jax.experimental.pallas import tpu_sc as plsc


_BLK = 1024

_NC = 2
_NS = 16
_NW = _NC * _NS
_LANES = 16
_CH = 32


def _tc_body(x_ref, pos_ref, o_ref):
    o_ref[...] = x_ref[...] + pos_ref[...][None, :, :]


def _tc_call(x, pos_emb):
    batch, length, dim = x.shape
    num_blocks = length // _BLK
    return pl.pallas_call(
        _tc_body,
        grid=(num_blocks,),
        in_specs=[
            pl.BlockSpec((batch, _BLK, dim), lambda i: (0, i, 0)),
            pl.BlockSpec((_BLK, dim), lambda i: (i, 0)),
        ],
        out_specs=pl.BlockSpec((batch, _BLK, dim), lambda i: (0, i, 0)),
        out_shape=jax.ShapeDtypeStruct(x.shape, x.dtype),
        compiler_params=pltpu.CompilerParams(
            dimension_semantics=("parallel",),
        ),
    )(x, pos_emb)


def _sc_body(batch, length, dim, x_hbm, pos_hbm, out_hbm, x_buf, pos_buf):
    wid = lax.axis_index("s") * _NC + lax.axis_index("c")
    seq_per_w = length // _NW
    chunk = _CH * dim
    n_chunks = seq_per_w // _CH

    def do_chunk(c, _):
        seq_base = (wid * seq_per_w + c * _CH) * dim
        pltpu.sync_copy(pos_hbm.at[pl.ds(seq_base, chunk)], pos_buf)
        for b in range(batch):
            x_base = b * length * dim + seq_base
            pltpu.sync_copy(x_hbm.at[pl.ds(x_base, chunk)], x_buf)

            def add16(j, _):
                sl = pl.ds(j * _LANES, _LANES)
                x_buf[sl] = x_buf[sl] + pos_buf[sl]
                return 0

            lax.fori_loop(0, chunk // _LANES, add16, 0)
            pltpu.sync_copy(x_buf, out_hbm.at[pl.ds(x_base, chunk)])
        return 0

    lax.fori_loop(0, n_chunks, do_chunk, 0)


def _sc_call(x, pos_emb):
    batch, length, dim = x.shape
    chunk = _CH * dim
    body = functools.partial(_sc_body, batch, length, dim)
    run = pl.kernel(
        body,
        out_type=jax.ShapeDtypeStruct((batch * length * dim,), x.dtype),
        mesh=plsc.VectorSubcoreMesh(core_axis_name="c", subcore_axis_name="s"),
        scratch_types=[
            pltpu.VMEM((chunk,), jnp.float32),
            pltpu.VMEM((chunk,), jnp.float32),
        ],
    )
    out = run(x.reshape(-1), pos_emb.reshape(-1))
    return out.reshape(x.shape)


def kernel(x, pos_emb):
    return _sc_call(x, pos_emb)


# speedup vs baseline: 1.7409x; 1.7409x over previous
import functools

import jax
import jax.numpy as jnp
from jax import lax
from jax.experimental import pallas as pl
from jax.experimental.pallas import tpu as pltpu
from jax.experimental.pallas import tpu_sc as plsc


_BLK = 1024

_NC = 2
_NS = 16
_NW = _NC * _NS
_LANES = 16
_CH = 8


def _tc_body(x_ref, pos_ref, o_ref):
    o_ref[...] = x_ref[...] + pos_ref[...][None, :, :]


def _tc_call(x, pos_emb):
    batch, length, dim = x.shape
    num_blocks = length // _BLK
    return pl.pallas_call(
        _tc_body,
        grid=(num_blocks,),
        in_specs=[
            pl.BlockSpec((batch, _BLK, dim), lambda i: (0, i, 0)),
            pl.BlockSpec((_BLK, dim), lambda i: (i, 0)),
        ],
        out_specs=pl.BlockSpec((batch, _BLK, dim), lambda i: (0, i, 0)),
        out_shape=jax.ShapeDtypeStruct(x.shape, x.dtype),
        compiler_params=pltpu.CompilerParams(
            dimension_semantics=("parallel",),
        ),
    )(x, pos_emb)


_UNROLL = 4


def _sc_body(batch, length, dim, x_hbm, pos_hbm, out_hbm,
             p0, p1, x0, x1, st0, st1, is0, is1, os0, os1):
    wid = lax.axis_index("s") * _NC + lax.axis_index("c")
    seq_per_w = length // _NW
    ce = _CH * dim
    n_chunks = seq_per_w // _CH
    pbufs, xbufs, stbufs = (p0, p1), (x0, x1), (st0, st1)
    isems, osems = (is0, is1), (os0, os1)

    def issue_in(c):
        s = c % 2
        seqoff = (wid * seq_per_w + c * _CH) * dim
        hs = [pltpu.async_copy(pos_hbm.at[pl.ds(seqoff, ce)], pbufs[s], isems[s])]
        for b in range(batch):
            hs.append(pltpu.async_copy(
                x_hbm.at[pl.ds(b * length * dim + seqoff, ce)],
                xbufs[s].at[pl.ds(b * ce, ce)], isems[s]))
        return hs

    def issue_out(c):
        s = c % 2
        seqoff = (wid * seq_per_w + c * _CH) * dim
        return [pltpu.async_copy(
            stbufs[s].at[pl.ds(b * ce, ce)],
            out_hbm.at[pl.ds(b * length * dim + seqoff, ce)], osems[s])
            for b in range(batch)]

    in_h = {0: issue_in(0), 1: issue_in(1)}
    out_h = {}
    for c in range(n_chunks):
        s = c % 2
        for h in in_h.pop(c):
            h.wait()
        if c - 2 >= 0:
            for h in out_h.pop(c - 2):
                h.wait()

        def cbody(j0, _):
            for dj in range(_UNROLL):
                j = j0 * _UNROLL + dj
                pv = pbufs[s][pl.ds(j * _LANES, _LANES)]
                for b in range(batch):
                    slb = pl.ds(b * ce + j * _LANES, _LANES)
                    stbufs[s][slb] = xbufs[s][slb] + pv
            return 0

        lax.fori_loop(0, ce // _LANES // _UNROLL, cbody, 0)
        out_h[c] = issue_out(c)
        if c + 2 < n_chunks:
            in_h[c + 2] = issue_in(c + 2)
    for c in (n_chunks - 2, n_chunks - 1):
        for h in out_h.pop(c):
            h.wait()


def _sc_call(x, pos_emb):
    batch, length, dim = x.shape
    ce = _CH * dim
    body = functools.partial(_sc_body, batch, length, dim)
    run = pl.kernel(
        body,
        out_type=jax.ShapeDtypeStruct((batch * length * dim,), x.dtype),
        mesh=plsc.VectorSubcoreMesh(core_axis_name="c", subcore_axis_name="s"),
        scratch_types=[
            pltpu.VMEM((ce,), jnp.float32),
            pltpu.VMEM((ce,), jnp.float32),
            pltpu.VMEM((batch * ce,), jnp.float32),
            pltpu.VMEM((batch * ce,), jnp.float32),
            pltpu.VMEM((batch * ce,), jnp.float32),
            pltpu.VMEM((batch * ce,), jnp.float32),
            pltpu.SemaphoreType.DMA,
            pltpu.SemaphoreType.DMA,
            pltpu.SemaphoreType.DMA,
            pltpu.SemaphoreType.DMA,
        ],
    )
    out = run(x.reshape(-1), pos_emb.reshape(-1))
    return out.reshape(x.shape)


def kernel(x, pos_emb):
    return _sc_call(x, pos_emb)


# speedup vs baseline: 8.0402x; 4.6185x over previous
import jax
import jax.numpy as jnp
from jax.experimental import pallas as pl
from jax.experimental.pallas import tpu as pltpu


_BLK = 512


def _add_kernel(x_ref, pos_ref, o_ref):
    o_ref[...] = x_ref[...] + pos_ref[...][None, :, :]


def kernel(x, pos_emb):
    batch, length, dim = x.shape
    num_blocks = length // _BLK
    return pl.pallas_call(
        _add_kernel,
        grid=(num_blocks,),
        in_specs=[
            pl.BlockSpec((batch, _BLK, dim), lambda i: (0, i, 0)),
            pl.BlockSpec((_BLK, dim), lambda i: (i, 0)),
        ],
        out_specs=pl.BlockSpec((batch, _BLK, dim), lambda i: (0, i, 0)),
        out_shape=jax.ShapeDtypeStruct(x.shape, x.dtype),
        compiler_params=pltpu.CompilerParams(
            dimension_semantics=("parallel",),
        ),
    )(x, pos_emb)


# speedup vs baseline: 8.0861x; 1.0057x over previous
import jax
import jax.numpy as jnp
from jax.experimental import pallas as pl
from jax.experimental.pallas import tpu as pltpu


_BLK = 1024


def _add_kernel(x_ref, pos_ref, o_ref):
    o_ref[...] = x_ref[...] + pos_ref[...][None, :, :]


def kernel(x, pos_emb):
    batch, length, dim = x.shape
    num_blocks = length // _BLK
    return pl.pallas_call(
        _add_kernel,
        grid=(num_blocks,),
        in_specs=[
            pl.BlockSpec((batch, _BLK, dim), lambda i: (0, i, 0)),
            pl.BlockSpec((_BLK, dim), lambda i: (i, 0)),
        ],
        out_specs=pl.BlockSpec((batch, _BLK, dim), lambda i: (0, i, 0)),
        out_shape=jax.ShapeDtypeStruct(x.shape, x.dtype),
        compiler_params=pltpu.CompilerParams(
            dimension_semantics=("parallel",),
        ),
    )(x, pos_emb)
